# baseline (speedup 1.0000x reference)
import functools

import jax
import jax.numpy as jnp
from jax import lax
from jax.experimental import pallas as pl
from jax.experimental.pallas import tpu as pltpu
from jax.experimental.pallas import tpu_sc as plsc

_NUM_CORES = 2
_NUM_SUBCORES = 16
_NUM_WORKERS = _NUM_CORES * _NUM_SUBCORES
_CHUNK = 128


@functools.lru_cache(maxsize=None)
def _make_sc_gather(E, N, H):
    n_chunks = E // _CHUNK
    iters = -(-n_chunks // _NUM_WORKERS)
    mesh = plsc.VectorSubcoreMesh(core_axis_name="c", subcore_axis_name="s")
    f32 = jnp.float32

    @functools.partial(
        pl.kernel,
        mesh=mesh,
        out_type=[
            jax.ShapeDtypeStruct((E, H), f32),
            jax.ShapeDtypeStruct((E, H), f32),
            jax.ShapeDtypeStruct((E,), f32),
            jax.ShapeDtypeStruct((E,), f32),
        ],
        scratch_types=[
            pltpu.VMEM((_CHUNK,), jnp.int32),
            pltpu.VMEM((_CHUNK,), jnp.int32),
            pltpu.VMEM((_CHUNK, H), f32),
            pltpu.VMEM((_CHUNK, H), f32),
            pltpu.VMEM((_CHUNK,), f32),
            pltpu.VMEM((_CHUNK,), f32),
            pltpu.VMEM((N,), f32),
            pltpu.SemaphoreType.DMA,
        ],
        compiler_params=pltpu.CompilerParams(needs_layout_passes=False),
    )
    def sc_gather(mem_hbm, lu_hbm, sidx_hbm, didx_hbm,
                  srows_hbm, drows_hbm, slu_hbm, dlu_hbm,
                  sidx_v, didx_v, srow_v, drow_v, slu_v, dlu_v, lu_v, sem):
        wid = lax.axis_index("s") * _NUM_CORES + lax.axis_index("c")
        pltpu.sync_copy(lu_hbm, lu_v)

        def body(i, _):
            chunk = wid + _NUM_WORKERS * i

            @pl.when(chunk < n_chunks)
            def _():
                base = chunk * _CHUNK
                pltpu.sync_copy(sidx_hbm.at[pl.ds(base, _CHUNK)], sidx_v)
                pltpu.sync_copy(didx_hbm.at[pl.ds(base, _CHUNK)], didx_v)
                cps = [
                    pltpu.async_copy(mem_hbm.at[sidx_v], srow_v, sem),
                    pltpu.async_copy(mem_hbm.at[didx_v], drow_v, sem),
                ]
                for j in range(_CHUNK // 16):
                    sl = pl.ds(j * 16, 16)
                    slu_v[sl] = plsc.load_gather(lu_v, [sidx_v[sl]])
                    dlu_v[sl] = plsc.load_gather(lu_v, [didx_v[sl]])
                for cp in cps:
                    cp.wait()
                pltpu.sync_copy(srow_v, srows_hbm.at[pl.ds(base, _CHUNK)])
                pltpu.sync_copy(drow_v, drows_hbm.at[pl.ds(base, _CHUNK)])
                pltpu.sync_copy(slu_v, slu_hbm.at[pl.ds(base, _CHUNK)])
                pltpu.sync_copy(dlu_v, dlu_hbm.at[pl.ds(base, _CHUNK)])

            return None

        lax.fori_loop(0, iters, body, None)

    return sc_gather


def _fast_cos(x):
    inv2pi = jnp.float32(0.15915493667125702)
    magic = jnp.float32(12582912.0)
    c1 = jnp.float32(6.28125)
    c2 = jnp.float32(0.0019353071795864769)
    k = (x * inv2pi + magic) - magic
    r = (x - k * c1) - k * c2
    u = r * r
    p = jnp.float32(1.736913401585966e-09)
    for c in (-2.711337329987122e-07, 2.47734242079983e-05,
              -0.0013887970411328634, 0.041666524363789405,
              -0.4999999177196379, 0.9999999922771011):
        p = p * u + jnp.float32(c)
    return p


def _assemble_body(type_ref, smask_ref, dmask_ref, ts_ref, slu_ref, dlu_ref,
                   srows_ref, drows_ref, evt_ref, w_ref, b_ref,
                   out_src_ref, out_dst_ref):
    H = srows_ref.shape[1]
    B = srows_ref.shape[0]
    type_col = type_ref[...].astype(jnp.float32)
    sm = smask_ref[...]
    dm = dmask_ref[...]
    t = ts_ref[...]
    w = w_ref[...]
    b = b_ref[...]
    src_embs = srows_ref[...] * sm
    dst_embs = drows_ref[...] * dm
    evt = evt_ref[...]
    src_te = _fast_cos((t - slu_ref[...] * dm) * w + b)
    dst_te = _fast_cos((t - dlu_ref[...] * dm) * w + b)
    type_b = jnp.broadcast_to(type_col, (B, H))

    out_src_ref[:, 0:H] = type_b
    out_src_ref[:, H:2 * H] = src_embs
    out_src_ref[:, 2 * H:3 * H] = dst_embs
    out_src_ref[:, 3 * H:4 * H] = src_te
    out_src_ref[:, 4 * H:5 * H] = evt

    out_dst_ref[:, 0:H] = type_b
    out_dst_ref[:, H:2 * H] = dst_embs
    out_dst_ref[:, 2 * H:3 * H] = src_embs
    out_dst_ref[:, 3 * H:4 * H] = dst_te
    out_dst_ref[:, 4 * H:5 * H] = evt


@functools.lru_cache(maxsize=None)
def _make_assemble(E, H, B=512, nblocks=None, block0=0, aliased=False,
                   interpret=False):
    if nblocks is None:
        nblocks = E // B
    col = pl.BlockSpec((B, 1), lambda i: (i, 0))
    row = pl.BlockSpec((B, H), lambda i: (i, 0))
    const = pl.BlockSpec((1, H), lambda i: (0, 0))
    in_specs = [col, col, col, col, col, col, row, row, row, const, const]
    if aliased:
        hbm = pl.BlockSpec(memory_space=pltpu.MemorySpace.HBM)
        in_specs += [hbm, hbm]

        def body(t, sm, dm, ts, sl, dl, sr, dr, ev, w, b, _a1, _a2, o1, o2):
            _assemble_body(t, sm, dm, ts, sl, dl, sr, dr, ev, w, b, o1, o2)
    else:
        body = _assemble_body
    return pl.pallas_call(
        body,
        grid=(nblocks,),
        in_specs=in_specs,
        out_specs=[
            pl.BlockSpec((B, 5 * H), lambda i: (block0 + i, 0)),
            pl.BlockSpec((B, 5 * H), lambda i: (block0 + i, 0)),
        ],
        out_shape=[
            jax.ShapeDtypeStruct((E, 5 * H), jnp.float32),
            jax.ShapeDtypeStruct((E, 5 * H), jnp.float32),
        ],
        input_output_aliases={11: 0, 12: 1} if aliased else {},
        compiler_params=pltpu.CompilerParams(
            dimension_semantics=("arbitrary",),
        ),
        interpret=interpret,
    )


_N_STAGES = 5
_BLOCK = 512


def kernel(event_type_ids, src_ids, src_mask, dst_ids, dst_mask,
           event_embeddings, event_timestamps, memory, last_update,
           time_w, time_b):
    E, H = event_embeddings.shape
    N = memory.shape[0]

    Ec = E // _N_STAGES
    nblocks = Ec // _BLOCK
    sc_gather = _make_sc_gather(Ec, N, H)
    src_ids = src_ids.astype(jnp.int32)
    dst_ids = dst_ids.astype(jnp.int32)
    type_col = event_type_ids.astype(jnp.int32).reshape(E, 1)
    smask = src_mask.reshape(E, 1)
    dmask = dst_mask.reshape(E, 1)
    ts = event_timestamps.reshape(E, 1)
    w = time_w.reshape(1, H)
    b = time_b.reshape(1, H)

    out_src = out_dst = None
    for c in range(_N_STAGES):
        lo = c * Ec
        srows, drows, slu, dlu = sc_gather(
            memory,
            last_update,
            lax.slice_in_dim(src_ids, lo, lo + Ec),
            lax.slice_in_dim(dst_ids, lo, lo + Ec),
        )
        args = (
            lax.slice_in_dim(type_col, lo, lo + Ec),
            lax.slice_in_dim(smask, lo, lo + Ec),
            lax.slice_in_dim(dmask, lo, lo + Ec),
            lax.slice_in_dim(ts, lo, lo + Ec),
            slu.reshape(Ec, 1),
            dlu.reshape(Ec, 1),
            srows,
            drows,
            lax.slice_in_dim(event_embeddings, lo, lo + Ec),
            w,
            b,
        )
        assemble = _make_assemble(E, H, _BLOCK, nblocks, c * nblocks,
                                  aliased=(c > 0))
        if c == 0:
            out_src, out_dst = assemble(*args)
        else:
            out_src, out_dst = assemble(*args, out_src, out_dst)
    return (out_src, out_dst)


# speedup vs baseline: 1.2807x; 1.2807x over previous
import functools

import jax
import jax.numpy as jnp
from jax import lax
from jax.experimental import pallas as pl
from jax.experimental.pallas import tpu as pltpu
from jax.experimental.pallas import tpu_sc as plsc

_NUM_CORES = 2
_NUM_SUBCORES = 16
_NUM_WORKERS = _NUM_CORES * _NUM_SUBCORES
_CHUNK = 128


@functools.lru_cache(maxsize=None)
def _make_sc_gather(E, N, H, lo=0):
    n_chunks = E // _CHUNK
    iters = -(-n_chunks // _NUM_WORKERS)
    mesh = plsc.VectorSubcoreMesh(core_axis_name="c", subcore_axis_name="s")
    f32 = jnp.float32

    @functools.partial(
        pl.kernel,
        mesh=mesh,
        out_type=[
            jax.ShapeDtypeStruct((E, H), f32),
            jax.ShapeDtypeStruct((E, H), f32),
            jax.ShapeDtypeStruct((E,), f32),
            jax.ShapeDtypeStruct((E,), f32),
        ],
        scratch_types=[
            pltpu.VMEM((_CHUNK,), jnp.int32),
            pltpu.VMEM((_CHUNK,), jnp.int32),
            pltpu.VMEM((_CHUNK, H), f32),
            pltpu.VMEM((_CHUNK, H), f32),
            pltpu.VMEM((_CHUNK,), f32),
            pltpu.VMEM((_CHUNK,), f32),
            pltpu.VMEM((N,), f32),
            pltpu.SemaphoreType.DMA,
        ],
        compiler_params=pltpu.CompilerParams(needs_layout_passes=False),
    )
    def sc_gather(mem_hbm, lu_hbm, sidx_hbm, didx_hbm,
                  srows_hbm, drows_hbm, slu_hbm, dlu_hbm,
                  sidx_v, didx_v, srow_v, drow_v, slu_v, dlu_v, lu_v, sem):
        wid = lax.axis_index("s") * _NUM_CORES + lax.axis_index("c")
        pltpu.sync_copy(lu_hbm, lu_v)

        def body(i, _):
            chunk = wid + _NUM_WORKERS * i

            @pl.when(chunk < n_chunks)
            def _():
                base = chunk * _CHUNK
                pltpu.sync_copy(sidx_hbm.at[pl.ds(lo + base, _CHUNK)], sidx_v)
                pltpu.sync_copy(didx_hbm.at[pl.ds(lo + base, _CHUNK)], didx_v)
                cps = [
                    pltpu.async_copy(mem_hbm.at[sidx_v], srow_v, sem),
                    pltpu.async_copy(mem_hbm.at[didx_v], drow_v, sem),
                ]
                for j in range(_CHUNK // 16):
                    sl = pl.ds(j * 16, 16)
                    slu_v[sl] = plsc.load_gather(lu_v, [sidx_v[sl]])
                    dlu_v[sl] = plsc.load_gather(lu_v, [didx_v[sl]])
                for cp in cps:
                    cp.wait()
                pltpu.sync_copy(srow_v, srows_hbm.at[pl.ds(base, _CHUNK)])
                pltpu.sync_copy(drow_v, drows_hbm.at[pl.ds(base, _CHUNK)])
                pltpu.sync_copy(slu_v, slu_hbm.at[pl.ds(base, _CHUNK)])
                pltpu.sync_copy(dlu_v, dlu_hbm.at[pl.ds(base, _CHUNK)])

            return None

        lax.fori_loop(0, iters, body, None)

    return sc_gather


def _fast_cos(x):
    inv2pi = jnp.float32(0.15915493667125702)
    magic = jnp.float32(12582912.0)
    c1 = jnp.float32(6.28125)
    c2 = jnp.float32(0.0019353071795864769)
    k = (x * inv2pi + magic) - magic
    r = (x - k * c1) - k * c2
    u = r * r
    p = jnp.float32(1.736913401585966e-09)
    for c in (-2.711337329987122e-07, 2.47734242079983e-05,
              -0.0013887970411328634, 0.041666524363789405,
              -0.4999999177196379, 0.9999999922771011):
        p = p * u + jnp.float32(c)
    return p


def _assemble_body(type_ref, smask_ref, dmask_ref, ts_ref, slu_ref, dlu_ref,
                   srows_ref, drows_ref, evt_ref, w_ref, b_ref,
                   out_src_ref, out_dst_ref):
    H = srows_ref.shape[1]
    B = srows_ref.shape[0]
    type_col = type_ref[...].astype(jnp.float32)
    sm = smask_ref[...]
    dm = dmask_ref[...]
    t = ts_ref[...]
    w = w_ref[...]
    b = b_ref[...]
    src_embs = srows_ref[...] * sm
    dst_embs = drows_ref[...] * dm
    evt = evt_ref[...]
    src_te = _fast_cos((t - slu_ref[...] * dm) * w + b)
    dst_te = _fast_cos((t - dlu_ref[...] * dm) * w + b)
    type_b = jnp.broadcast_to(type_col, (B, H))

    out_src_ref[:, 0:H] = type_b
    out_src_ref[:, H:2 * H] = src_embs
    out_src_ref[:, 2 * H:3 * H] = dst_embs
    out_src_ref[:, 3 * H:4 * H] = src_te
    out_src_ref[:, 4 * H:5 * H] = evt

    out_dst_ref[:, 0:H] = type_b
    out_dst_ref[:, H:2 * H] = dst_embs
    out_dst_ref[:, 2 * H:3 * H] = src_embs
    out_dst_ref[:, 3 * H:4 * H] = dst_te
    out_dst_ref[:, 4 * H:5 * H] = evt


@functools.lru_cache(maxsize=None)
def _make_assemble(E, H, B=512, nblocks=None, block0=0, aliased=False,
                   interpret=False):
    if nblocks is None:
        nblocks = E // B
    col = pl.BlockSpec((B, 1), lambda i: (block0 + i, 0))
    ccol = pl.BlockSpec((B, 1), lambda i: (i, 0))
    crow = pl.BlockSpec((B, H), lambda i: (i, 0))
    row = pl.BlockSpec((B, H), lambda i: (block0 + i, 0))
    const = pl.BlockSpec((1, H), lambda i: (0, 0))
    in_specs = [col, col, col, col, ccol, ccol, crow, crow, row, const, const]
    if aliased:
        hbm = pl.BlockSpec(memory_space=pltpu.MemorySpace.HBM)
        in_specs += [hbm, hbm]

        def body(t, sm, dm, ts, sl, dl, sr, dr, ev, w, b, _a1, _a2, o1, o2):
            _assemble_body(t, sm, dm, ts, sl, dl, sr, dr, ev, w, b, o1, o2)
    else:
        body = _assemble_body
    return pl.pallas_call(
        body,
        grid=(nblocks,),
        in_specs=in_specs,
        out_specs=[
            pl.BlockSpec((B, 5 * H), lambda i: (block0 + i, 0)),
            pl.BlockSpec((B, 5 * H), lambda i: (block0 + i, 0)),
        ],
        out_shape=[
            jax.ShapeDtypeStruct((E, 5 * H), jnp.float32),
            jax.ShapeDtypeStruct((E, 5 * H), jnp.float32),
        ],
        input_output_aliases={11: 0, 12: 1} if aliased else {},
        compiler_params=pltpu.CompilerParams(
            dimension_semantics=("arbitrary",),
        ),
        interpret=interpret,
    )


_N_STAGES = 5
_BLOCK = 512


def kernel(event_type_ids, src_ids, src_mask, dst_ids, dst_mask,
           event_embeddings, event_timestamps, memory, last_update,
           time_w, time_b):
    E, H = event_embeddings.shape
    N = memory.shape[0]

    Ec = E // _N_STAGES
    nblocks = Ec // _BLOCK
    src_ids = src_ids.astype(jnp.int32)
    dst_ids = dst_ids.astype(jnp.int32)
    type_col = event_type_ids.astype(jnp.int32).reshape(E, 1)
    smask = src_mask.reshape(E, 1)
    dmask = dst_mask.reshape(E, 1)
    ts = event_timestamps.reshape(E, 1)
    w = time_w.reshape(1, H)
    b = time_b.reshape(1, H)

    out_src = out_dst = None
    for c in range(_N_STAGES):
        lo = c * Ec
        srows, drows, slu, dlu = _make_sc_gather(Ec, N, H, lo)(
            memory,
            last_update,
            src_ids,
            dst_ids,
        )
        args = (
            type_col,
            smask,
            dmask,
            ts,
            slu.reshape(Ec, 1),
            dlu.reshape(Ec, 1),
            srows,
            drows,
            event_embeddings,
            w,
            b,
        )
        assemble = _make_assemble(E, H, _BLOCK, nblocks, c * nblocks,
                                  aliased=(c > 0))
        if c == 0:
            out_src, out_dst = assemble(*args)
        else:
            out_src, out_dst = assemble(*args, out_src, out_dst)
    return (out_src, out_dst)


# speedup vs baseline: 1.3750x; 1.0736x over previous
import functools

import jax
import jax.numpy as jnp
from jax import lax
from jax.experimental import pallas as pl
from jax.experimental.pallas import tpu as pltpu
from jax.experimental.pallas import tpu_sc as plsc

_NUM_CORES = 2
_NUM_SUBCORES = 16
_NUM_WORKERS = _NUM_CORES * _NUM_SUBCORES
_CHUNK = 128


@functools.lru_cache(maxsize=None)
def _make_sc_gather(E, N, H, lo=0):
    n_chunks = E // _CHUNK
    iters = -(-n_chunks // _NUM_WORKERS)
    mesh = plsc.VectorSubcoreMesh(core_axis_name="c", subcore_axis_name="s")
    f32 = jnp.float32

    @functools.partial(
        pl.kernel,
        mesh=mesh,
        out_type=[
            jax.ShapeDtypeStruct((E, H), f32),
            jax.ShapeDtypeStruct((E, H), f32),
            jax.ShapeDtypeStruct((E,), f32),
            jax.ShapeDtypeStruct((E,), f32),
        ],
        scratch_types=[
            pltpu.VMEM((_CHUNK,), jnp.int32),
            pltpu.VMEM((_CHUNK,), jnp.int32),
            pltpu.VMEM((_CHUNK, H), f32),
            pltpu.VMEM((_CHUNK, H), f32),
            pltpu.VMEM((_CHUNK,), f32),
            pltpu.VMEM((_CHUNK,), f32),
            pltpu.VMEM((N,), f32),
            pltpu.SemaphoreType.DMA,
        ],
        compiler_params=pltpu.CompilerParams(needs_layout_passes=False),
    )
    def sc_gather(mem_hbm, lu_hbm, sidx_hbm, didx_hbm,
                  srows_hbm, drows_hbm, slu_hbm, dlu_hbm,
                  sidx_v, didx_v, srow_v, drow_v, slu_v, dlu_v, lu_v, sem):
        wid = lax.axis_index("s") * _NUM_CORES + lax.axis_index("c")
        pltpu.sync_copy(lu_hbm, lu_v)

        def body(i, _):
            chunk = wid + _NUM_WORKERS * i

            @pl.when(chunk < n_chunks)
            def _():
                base = chunk * _CHUNK
                pltpu.sync_copy(sidx_hbm.at[pl.ds(lo + base, _CHUNK)], sidx_v)
                pltpu.sync_copy(didx_hbm.at[pl.ds(lo + base, _CHUNK)], didx_v)
                cps = [
                    pltpu.async_copy(mem_hbm.at[sidx_v], srow_v, sem),
                    pltpu.async_copy(mem_hbm.at[didx_v], drow_v, sem),
                ]
                for j in range(_CHUNK // 16):
                    sl = pl.ds(j * 16, 16)
                    slu_v[sl] = plsc.load_gather(lu_v, [sidx_v[sl]])
                    dlu_v[sl] = plsc.load_gather(lu_v, [didx_v[sl]])
                for cp in cps:
                    cp.wait()
                pltpu.sync_copy(srow_v, srows_hbm.at[pl.ds(base, _CHUNK)])
                pltpu.sync_copy(drow_v, drows_hbm.at[pl.ds(base, _CHUNK)])
                pltpu.sync_copy(slu_v, slu_hbm.at[pl.ds(base, _CHUNK)])
                pltpu.sync_copy(dlu_v, dlu_hbm.at[pl.ds(base, _CHUNK)])

            return None

        lax.fori_loop(0, iters, body, None)

    return sc_gather


def _fast_cos(x):
    inv2pi = jnp.float32(0.15915493667125702)
    magic = jnp.float32(12582912.0)
    c1 = jnp.float32(6.28125)
    c2 = jnp.float32(0.0019353071795864769)
    k = (x * inv2pi + magic) - magic
    r = (x - k * c1) - k * c2
    u = r * r
    p = jnp.float32(1.736913401585966e-09)
    for c in (-2.711337329987122e-07, 2.47734242079983e-05,
              -0.0013887970411328634, 0.041666524363789405,
              -0.4999999177196379, 0.9999999922771011):
        p = p * u + jnp.float32(c)
    return p


def _assemble_body(type_ref, smask_ref, dmask_ref, ts_ref, slu_ref, dlu_ref,
                   srows_ref, drows_ref, evt_ref, w_ref, b_ref,
                   out_src_ref, out_dst_ref):
    H = srows_ref.shape[1]
    B = srows_ref.shape[0]
    type_col = type_ref[...].astype(jnp.float32)
    sm = smask_ref[...]
    dm = dmask_ref[...]
    t = ts_ref[...]
    w = w_ref[...]
    b = b_ref[...]
    src_embs = srows_ref[...] * sm
    dst_embs = drows_ref[...] * dm
    evt = evt_ref[...]
    src_te = _fast_cos((t - slu_ref[...] * dm) * w + b)
    dst_te = _fast_cos((t - dlu_ref[...] * dm) * w + b)
    type_b = jnp.broadcast_to(type_col, (B, H))

    out_src_ref[:, 0:H] = type_b
    out_src_ref[:, H:2 * H] = src_embs
    out_src_ref[:, 2 * H:3 * H] = dst_embs
    out_src_ref[:, 3 * H:4 * H] = src_te
    out_src_ref[:, 4 * H:5 * H] = evt

    out_dst_ref[:, 0:H] = type_b
    out_dst_ref[:, H:2 * H] = dst_embs
    out_dst_ref[:, 2 * H:3 * H] = src_embs
    out_dst_ref[:, 3 * H:4 * H] = dst_te
    out_dst_ref[:, 4 * H:5 * H] = evt


@functools.lru_cache(maxsize=None)
def _make_assemble(E, H, B=512, nblocks=None, block0=0, aliased=False,
                   interpret=False):
    if nblocks is None:
        nblocks = E // B
    col = pl.BlockSpec((B, 1), lambda i: (block0 + i, 0))
    ccol = pl.BlockSpec((B, 1), lambda i: (i, 0))
    crow = pl.BlockSpec((B, H), lambda i: (i, 0))
    row = pl.BlockSpec((B, H), lambda i: (block0 + i, 0))
    const = pl.BlockSpec((1, H), lambda i: (0, 0))
    in_specs = [col, col, col, col, ccol, ccol, crow, crow, row, const, const]
    if aliased:
        hbm = pl.BlockSpec(memory_space=pltpu.MemorySpace.HBM)
        in_specs += [hbm, hbm]

        def body(t, sm, dm, ts, sl, dl, sr, dr, ev, w, b, _a1, _a2, o1, o2):
            _assemble_body(t, sm, dm, ts, sl, dl, sr, dr, ev, w, b, o1, o2)
    else:
        body = _assemble_body
    return pl.pallas_call(
        body,
        grid=(nblocks,),
        in_specs=in_specs,
        out_specs=[
            pl.BlockSpec((B, 5 * H), lambda i: (block0 + i, 0)),
            pl.BlockSpec((B, 5 * H), lambda i: (block0 + i, 0)),
        ],
        out_shape=[
            jax.ShapeDtypeStruct((E, 5 * H), jnp.float32),
            jax.ShapeDtypeStruct((E, 5 * H), jnp.float32),
        ],
        input_output_aliases={11: 0, 12: 1} if aliased else {},
        compiler_params=pltpu.CompilerParams(
            dimension_semantics=("arbitrary",),
        ),
        interpret=interpret,
    )


_N_STAGES = 1
_BLOCK = 1024


def kernel(event_type_ids, src_ids, src_mask, dst_ids, dst_mask,
           event_embeddings, event_timestamps, memory, last_update,
           time_w, time_b):
    E, H = event_embeddings.shape
    N = memory.shape[0]

    Ec = E // _N_STAGES
    nblocks = Ec // _BLOCK
    src_ids = src_ids.astype(jnp.int32)
    dst_ids = dst_ids.astype(jnp.int32)
    type_col = event_type_ids.astype(jnp.int32).reshape(E, 1)
    smask = src_mask.reshape(E, 1)
    dmask = dst_mask.reshape(E, 1)
    ts = event_timestamps.reshape(E, 1)
    w = time_w.reshape(1, H)
    b = time_b.reshape(1, H)

    out_src = out_dst = None
    for c in range(_N_STAGES):
        lo = c * Ec
        srows, drows, slu, dlu = _make_sc_gather(Ec, N, H, lo)(
            memory,
            last_update,
            src_ids,
            dst_ids,
        )
        args = (
            type_col,
            smask,
            dmask,
            ts,
            slu.reshape(Ec, 1),
            dlu.reshape(Ec, 1),
            srows,
            drows,
            event_embeddings,
            w,
            b,
        )
        assemble = _make_assemble(E, H, _BLOCK, nblocks, c * nblocks,
                                  aliased=(c > 0))
        if c == 0:
            out_src, out_dst = assemble(*args)
        else:
            out_src, out_dst = assemble(*args, out_src, out_dst)
    return (out_src, out_dst)


# speedup vs baseline: 1.3858x; 1.0079x over previous
import functools

import jax
import jax.numpy as jnp
from jax import lax
from jax.experimental import pallas as pl
from jax.experimental.pallas import tpu as pltpu
from jax.experimental.pallas import tpu_sc as plsc

_NUM_CORES = 2
_NUM_SUBCORES = 16
_NUM_WORKERS = _NUM_CORES * _NUM_SUBCORES
_CHUNK = 128


@functools.lru_cache(maxsize=None)
def _make_sc_gather(E, N, H, lo=0):
    n_chunks = E // _CHUNK
    iters = -(-n_chunks // _NUM_WORKERS)
    mesh = plsc.VectorSubcoreMesh(core_axis_name="c", subcore_axis_name="s")
    f32 = jnp.float32

    @functools.partial(
        pl.kernel,
        mesh=mesh,
        out_type=[
            jax.ShapeDtypeStruct((E, H), f32),
            jax.ShapeDtypeStruct((E, H), f32),
            jax.ShapeDtypeStruct((E,), f32),
            jax.ShapeDtypeStruct((E,), f32),
        ],
        scratch_types=[
            pltpu.VMEM((_CHUNK,), jnp.int32),
            pltpu.VMEM((_CHUNK,), jnp.int32),
            pltpu.VMEM((_CHUNK, H), f32),
            pltpu.VMEM((_CHUNK, H), f32),
            pltpu.VMEM((_CHUNK,), f32),
            pltpu.VMEM((_CHUNK,), f32),
            pltpu.VMEM((N,), f32),
            pltpu.SemaphoreType.DMA,
        ],
        compiler_params=pltpu.CompilerParams(needs_layout_passes=False),
    )
    def sc_gather(mem_hbm, lu_hbm, sidx_hbm, didx_hbm,
                  srows_hbm, drows_hbm, slu_hbm, dlu_hbm,
                  sidx_v, didx_v, srow_v, drow_v, slu_v, dlu_v, lu_v, sem):
        wid = lax.axis_index("s") * _NUM_CORES + lax.axis_index("c")
        pltpu.sync_copy(lu_hbm, lu_v)

        def body(i, _):
            chunk = wid + _NUM_WORKERS * i

            @pl.when(chunk < n_chunks)
            def _():
                base = chunk * _CHUNK
                pltpu.sync_copy(sidx_hbm.at[pl.ds(lo + base, _CHUNK)], sidx_v)
                pltpu.sync_copy(didx_hbm.at[pl.ds(lo + base, _CHUNK)], didx_v)
                cps = [
                    pltpu.async_copy(mem_hbm.at[sidx_v], srow_v, sem),
                    pltpu.async_copy(mem_hbm.at[didx_v], drow_v, sem),
                ]
                for j in range(_CHUNK // 16):
                    sl = pl.ds(j * 16, 16)
                    slu_v[sl] = plsc.load_gather(lu_v, [sidx_v[sl]])
                    dlu_v[sl] = plsc.load_gather(lu_v, [didx_v[sl]])
                for cp in cps:
                    cp.wait()
                pltpu.sync_copy(srow_v, srows_hbm.at[pl.ds(base, _CHUNK)])
                pltpu.sync_copy(drow_v, drows_hbm.at[pl.ds(base, _CHUNK)])
                pltpu.sync_copy(slu_v, slu_hbm.at[pl.ds(base, _CHUNK)])
                pltpu.sync_copy(dlu_v, dlu_hbm.at[pl.ds(base, _CHUNK)])

            return None

        lax.fori_loop(0, iters, body, None)

    return sc_gather


def _fast_cos(x):
    inv2pi = jnp.float32(0.15915493667125702)
    magic = jnp.float32(12582912.0)
    c1 = jnp.float32(6.28125)
    c2 = jnp.float32(0.0019353071795864769)
    k = (x * inv2pi + magic) - magic
    r = (x - k * c1) - k * c2
    u = r * r
    p = jnp.float32(1.736913401585966e-09)
    for c in (-2.711337329987122e-07, 2.47734242079983e-05,
              -0.0013887970411328634, 0.041666524363789405,
              -0.4999999177196379, 0.9999999922771011):
        p = p * u + jnp.float32(c)
    return p


def _assemble_body(type_ref, smask_ref, dmask_ref, ts_ref, slu_ref, dlu_ref,
                   srows_ref, drows_ref, evt_ref, w_ref, b_ref,
                   out_src_ref, out_dst_ref):
    H = srows_ref.shape[1]
    B = srows_ref.shape[0]
    type_col = type_ref[...].astype(jnp.float32)
    sm = smask_ref[...]
    dm = dmask_ref[...]
    t = ts_ref[...]
    w = w_ref[...]
    b = b_ref[...]
    src_embs = srows_ref[...] * sm
    dst_embs = drows_ref[...] * dm
    evt = evt_ref[...]
    src_te = _fast_cos((t - slu_ref[...] * dm) * w + b)
    dst_te = _fast_cos((t - dlu_ref[...] * dm) * w + b)
    type_b = jnp.broadcast_to(type_col, (B, H))

    out_src_ref[:, 0:H] = type_b
    out_src_ref[:, H:2 * H] = src_embs
    out_src_ref[:, 2 * H:3 * H] = dst_embs
    out_src_ref[:, 3 * H:4 * H] = src_te
    out_src_ref[:, 4 * H:5 * H] = evt

    out_dst_ref[:, 0:H] = type_b
    out_dst_ref[:, H:2 * H] = dst_embs
    out_dst_ref[:, 2 * H:3 * H] = src_embs
    out_dst_ref[:, 3 * H:4 * H] = dst_te
    out_dst_ref[:, 4 * H:5 * H] = evt


@functools.lru_cache(maxsize=None)
def _make_assemble(E, H, B=512, nblocks=None, block0=0, aliased=False,
                   interpret=False):
    if nblocks is None:
        nblocks = E // B
    col = pl.BlockSpec((B, 1), lambda i: (block0 + i, 0))
    ccol = pl.BlockSpec((B, 1), lambda i: (i, 0))
    crow = pl.BlockSpec((B, H), lambda i: (i, 0))
    row = pl.BlockSpec((B, H), lambda i: (block0 + i, 0))
    const = pl.BlockSpec((1, H), lambda i: (0, 0))
    in_specs = [col, col, col, col, ccol, ccol, crow, crow, row, const, const]
    if aliased:
        hbm = pl.BlockSpec(memory_space=pltpu.MemorySpace.HBM)
        in_specs += [hbm, hbm]

        def body(t, sm, dm, ts, sl, dl, sr, dr, ev, w, b, _a1, _a2, o1, o2):
            _assemble_body(t, sm, dm, ts, sl, dl, sr, dr, ev, w, b, o1, o2)
    else:
        body = _assemble_body
    return pl.pallas_call(
        body,
        grid=(nblocks,),
        in_specs=in_specs,
        out_specs=[
            pl.BlockSpec((B, 5 * H), lambda i: (block0 + i, 0)),
            pl.BlockSpec((B, 5 * H), lambda i: (block0 + i, 0)),
        ],
        out_shape=[
            jax.ShapeDtypeStruct((E, 5 * H), jnp.float32),
            jax.ShapeDtypeStruct((E, 5 * H), jnp.float32),
        ],
        input_output_aliases={11: 0, 12: 1} if aliased else {},
        compiler_params=pltpu.CompilerParams(
            dimension_semantics=("arbitrary",),
        ),
        interpret=interpret,
    )


_N_STAGES = 1
_BLOCK = 1280


def kernel(event_type_ids, src_ids, src_mask, dst_ids, dst_mask,
           event_embeddings, event_timestamps, memory, last_update,
           time_w, time_b):
    E, H = event_embeddings.shape
    N = memory.shape[0]

    Ec = E // _N_STAGES
    nblocks = Ec // _BLOCK
    src_ids = src_ids.astype(jnp.int32)
    dst_ids = dst_ids.astype(jnp.int32)
    type_col = event_type_ids.astype(jnp.int32).reshape(E, 1)
    smask = src_mask.reshape(E, 1)
    dmask = dst_mask.reshape(E, 1)
    ts = event_timestamps.reshape(E, 1)
    w = time_w.reshape(1, H)
    b = time_b.reshape(1, H)

    out_src = out_dst = None
    for c in range(_N_STAGES):
        lo = c * Ec
        srows, drows, slu, dlu = _make_sc_gather(Ec, N, H, lo)(
            memory,
            last_update,
            src_ids,
            dst_ids,
        )
        args = (
            type_col,
            smask,
            dmask,
            ts,
            slu.reshape(Ec, 1),
            dlu.reshape(Ec, 1),
            srows,
            drows,
            event_embeddings,
            w,
            b,
        )
        assemble = _make_assemble(E, H, _BLOCK, nblocks, c * nblocks,
                                  aliased=(c > 0))
        if c == 0:
            out_src, out_dst = assemble(*args)
        else:
            out_src, out_dst = assemble(*args, out_src, out_dst)
    return (out_src, out_dst)


# speedup vs baseline: 1.3897x; 1.0028x over previous
import functools

import jax
import jax.numpy as jnp
from jax import lax
from jax.experimental import pallas as pl
from jax.experimental.pallas import tpu as pltpu
from jax.experimental.pallas import tpu_sc as plsc

_NUM_CORES = 2
_NUM_SUBCORES = 16
_NUM_WORKERS = _NUM_CORES * _NUM_SUBCORES
_CHUNK = 128


@functools.lru_cache(maxsize=None)
def _make_sc_gather(E, N, H, lo=0):
    n_chunks = E // _CHUNK
    iters = -(-n_chunks // _NUM_WORKERS)
    mesh = plsc.VectorSubcoreMesh(core_axis_name="c", subcore_axis_name="s")
    f32 = jnp.float32

    @functools.partial(
        pl.kernel,
        mesh=mesh,
        out_type=[
            jax.ShapeDtypeStruct((E, H), f32),
            jax.ShapeDtypeStruct((E, H), f32),
            jax.ShapeDtypeStruct((E,), f32),
            jax.ShapeDtypeStruct((E,), f32),
        ],
        scratch_types=[
            pltpu.VMEM((_CHUNK,), jnp.int32),
            pltpu.VMEM((_CHUNK,), jnp.int32),
            pltpu.VMEM((_CHUNK, H), f32),
            pltpu.VMEM((_CHUNK, H), f32),
            pltpu.VMEM((_CHUNK,), f32),
            pltpu.VMEM((_CHUNK,), f32),
            pltpu.VMEM((N,), f32),
            pltpu.SemaphoreType.DMA,
        ],
        compiler_params=pltpu.CompilerParams(needs_layout_passes=False),
    )
    def sc_gather(mem_hbm, lu_hbm, sidx_hbm, didx_hbm,
                  srows_hbm, drows_hbm, slu_hbm, dlu_hbm,
                  sidx_v, didx_v, srow_v, drow_v, slu_v, dlu_v, lu_v, sem):
        wid = lax.axis_index("s") * _NUM_CORES + lax.axis_index("c")
        pltpu.sync_copy(lu_hbm, lu_v)

        def body(i, _):
            chunk = wid + _NUM_WORKERS * i

            @pl.when(chunk < n_chunks)
            def _():
                base = chunk * _CHUNK
                pltpu.sync_copy(sidx_hbm.at[pl.ds(lo + base, _CHUNK)], sidx_v)
                pltpu.sync_copy(didx_hbm.at[pl.ds(lo + base, _CHUNK)], didx_v)
                cps = [
                    pltpu.async_copy(mem_hbm.at[sidx_v], srow_v, sem),
                    pltpu.async_copy(mem_hbm.at[didx_v], drow_v, sem),
                ]
                for j in range(_CHUNK // 16):
                    sl = pl.ds(j * 16, 16)
                    slu_v[sl] = plsc.load_gather(lu_v, [sidx_v[sl]])
                    dlu_v[sl] = plsc.load_gather(lu_v, [didx_v[sl]])
                for cp in cps:
                    cp.wait()
                pltpu.sync_copy(srow_v, srows_hbm.at[pl.ds(base, _CHUNK)])
                pltpu.sync_copy(drow_v, drows_hbm.at[pl.ds(base, _CHUNK)])
                pltpu.sync_copy(slu_v, slu_hbm.at[pl.ds(base, _CHUNK)])
                pltpu.sync_copy(dlu_v, dlu_hbm.at[pl.ds(base, _CHUNK)])

            return None

        lax.fori_loop(0, iters, body, None)

    return sc_gather


def _fast_cos(x):
    inv2pi = jnp.float32(0.15915493667125702)
    magic = jnp.float32(12582912.0)
    c1 = jnp.float32(6.28125)
    c2 = jnp.float32(0.0019353071795864769)
    k = (x * inv2pi + magic) - magic
    r = (x - k * c1) - k * c2
    u = r * r
    p = jnp.float32(1.736913401585966e-09)
    for c in (-2.711337329987122e-07, 2.47734242079983e-05,
              -0.0013887970411328634, 0.041666524363789405,
              -0.4999999177196379, 0.9999999922771011):
        p = p * u + jnp.float32(c)
    return p


def _assemble_body(type_ref, smask_ref, dmask_ref, ts_ref, slu_ref, dlu_ref,
                   srows_ref, drows_ref, evt_ref, w_ref, b_ref,
                   out_src_ref, out_dst_ref):
    H = srows_ref.shape[1]
    B = srows_ref.shape[0]
    type_col = type_ref[...].astype(jnp.float32)
    sm = smask_ref[...]
    dm = dmask_ref[...]
    t = ts_ref[...]
    w = w_ref[...]
    b = b_ref[...]
    src_embs = srows_ref[...] * sm
    dst_embs = drows_ref[...] * dm
    evt = evt_ref[...]
    src_te = _fast_cos((t - slu_ref[...] * dm) * w + b)
    dst_te = _fast_cos((t - dlu_ref[...] * dm) * w + b)
    type_b = jnp.broadcast_to(type_col, (B, H))

    out_src_ref[:, 0:H] = type_b
    out_src_ref[:, H:2 * H] = src_embs
    out_src_ref[:, 2 * H:3 * H] = dst_embs
    out_src_ref[:, 3 * H:4 * H] = src_te
    out_src_ref[:, 4 * H:5 * H] = evt

    out_dst_ref[:, 0:H] = type_b
    out_dst_ref[:, H:2 * H] = dst_embs
    out_dst_ref[:, 2 * H:3 * H] = src_embs
    out_dst_ref[:, 3 * H:4 * H] = dst_te
    out_dst_ref[:, 4 * H:5 * H] = evt


@functools.lru_cache(maxsize=None)
def _make_assemble(E, H, B=512, nblocks=None, block0=0, aliased=False,
                   interpret=False):
    if nblocks is None:
        nblocks = E // B
    col = pl.BlockSpec((B, 1), lambda i: (block0 + i, 0))
    ccol = pl.BlockSpec((B, 1), lambda i: (i, 0))
    crow = pl.BlockSpec((B, H), lambda i: (i, 0))
    row = pl.BlockSpec((B, H), lambda i: (block0 + i, 0))
    const = pl.BlockSpec((1, H), lambda i: (0, 0))
    in_specs = [col, col, col, col, ccol, ccol, crow, crow, row, const, const]
    if aliased:
        hbm = pl.BlockSpec(memory_space=pltpu.MemorySpace.HBM)
        in_specs += [hbm, hbm]

        def body(t, sm, dm, ts, sl, dl, sr, dr, ev, w, b, _a1, _a2, o1, o2):
            _assemble_body(t, sm, dm, ts, sl, dl, sr, dr, ev, w, b, o1, o2)
    else:
        body = _assemble_body
    return pl.pallas_call(
        body,
        grid=(nblocks,),
        in_specs=in_specs,
        out_specs=[
            pl.BlockSpec((B, 5 * H), lambda i: (block0 + i, 0)),
            pl.BlockSpec((B, 5 * H), lambda i: (block0 + i, 0)),
        ],
        out_shape=[
            jax.ShapeDtypeStruct((E, 5 * H), jnp.float32),
            jax.ShapeDtypeStruct((E, 5 * H), jnp.float32),
        ],
        input_output_aliases={11: 0, 12: 1} if aliased else {},
        compiler_params=pltpu.CompilerParams(
            dimension_semantics=("arbitrary",),
        ),
        interpret=interpret,
    )


_N_STAGES = 1
_BLOCK = 1600


def kernel(event_type_ids, src_ids, src_mask, dst_ids, dst_mask,
           event_embeddings, event_timestamps, memory, last_update,
           time_w, time_b):
    E, H = event_embeddings.shape
    N = memory.shape[0]

    Ec = E // _N_STAGES
    nblocks = Ec // _BLOCK
    src_ids = src_ids.astype(jnp.int32)
    dst_ids = dst_ids.astype(jnp.int32)
    type_col = event_type_ids.astype(jnp.int32).reshape(E, 1)
    smask = src_mask.reshape(E, 1)
    dmask = dst_mask.reshape(E, 1)
    ts = event_timestamps.reshape(E, 1)
    w = time_w.reshape(1, H)
    b = time_b.reshape(1, H)

    out_src = out_dst = None
    for c in range(_N_STAGES):
        lo = c * Ec
        srows, drows, slu, dlu = _make_sc_gather(Ec, N, H, lo)(
            memory,
            last_update,
            src_ids,
            dst_ids,
        )
        args = (
            type_col,
            smask,
            dmask,
            ts,
            slu.reshape(Ec, 1),
            dlu.reshape(Ec, 1),
            srows,
            drows,
            event_embeddings,
            w,
            b,
        )
        assemble = _make_assemble(E, H, _BLOCK, nblocks, c * nblocks,
                                  aliased=(c > 0))
        if c == 0:
            out_src, out_dst = assemble(*args)
        else:
            out_src, out_dst = assemble(*args, out_src, out_dst)
    return (out_src, out_dst)
